# baseline (speedup 1.0000x reference)
import functools

import jax
import jax.numpy as jnp
from jax import lax
from jax.experimental import pallas as pl
from jax.experimental.pallas import tpu as pltpu
from jax.experimental.pallas import tpu_sc as plsc

D = 64
N_ROWS = 1000000
N_TRIP = 16384
T = 2 * N_TRIP

NC, NS = 2, 16
NW = NC * NS
TRIP_PER_W = T // NW
CHUNK = 256
N_CHUNK = TRIP_PER_W // CHUNK
STREAMS = CHUNK // 128
NBUF = 2


def _sc_atom(table, pred2, heads2d, tails2d):
    mesh = plsc.VectorSubcoreMesh(core_axis_name="c", subcore_axis_name="s")
    idx_rows = TRIP_PER_W // 128

    @functools.partial(
        pl.kernel,
        out_type=jax.ShapeDtypeStruct((T, D), jnp.float32),
        mesh=mesh,
        scratch_types=[
            pltpu.VMEM((idx_rows, 128), jnp.int32),
            pltpu.VMEM((idx_rows, 128), jnp.int32),
            pltpu.VMEM((D,), jnp.float32),
            pltpu.VMEM((NBUF, CHUNK, D), jnp.float32),
            pltpu.VMEM((NBUF, CHUNK, D), jnp.float32),
            pltpu.VMEM((CHUNK, D), jnp.float32),
            pltpu.SemaphoreType.DMA,
            pltpu.SemaphoreType.DMA,
        ],
        compiler_params=pltpu.CompilerParams(use_tc_tiling_on_sc=False),
    )
    def k(table_hbm, pred_hbm, heads_hbm, tails_hbm, out_hbm,
          hidx_v, tidx_v, pred_v, hrows_v, trows_v, atom_v, hsem, tsem):
        wid = lax.axis_index("s") * NC + lax.axis_index("c")
        base = wid * TRIP_PER_W
        pltpu.sync_copy(heads_hbm.at[pl.ds(wid * idx_rows, idx_rows)], hidx_v)
        pltpu.sync_copy(tails_hbm.at[pl.ds(wid * idx_rows, idx_rows)], tidx_v)
        pltpu.sync_copy(pred_hbm.at[wid // NS], pred_v)
        pk = [pred_v[pl.ds(16 * q, 16)] for q in range(D // 16)]

        def fire(c, buf):
            for j in range(STREAMS):
                pltpu.async_copy(
                    table_hbm.at[hidx_v.at[c * STREAMS + j]],
                    hrows_v.at[buf].at[pl.ds(j * 128, 128)], hsem)
                pltpu.async_copy(
                    table_hbm.at[tidx_v.at[c * STREAMS + j]],
                    trows_v.at[buf].at[pl.ds(j * 128, 128)], tsem)

        def drain(c, buf):
            for j in range(STREAMS):
                pltpu.make_async_copy(
                    table_hbm.at[hidx_v.at[c * STREAMS + j]],
                    hrows_v.at[buf].at[pl.ds(j * 128, 128)], hsem).wait()
                pltpu.make_async_copy(
                    table_hbm.at[tidx_v.at[c * STREAMS + j]],
                    trows_v.at[buf].at[pl.ds(j * 128, 128)], tsem).wait()

        fire(0, 0)
        for c in range(N_CHUNK):
            buf = c % NBUF
            drain(c, buf)
            if c + 1 < N_CHUNK:
                fire(c + 1, (c + 1) % NBUF)

            def rbody(r, carry):
                for q in range(D // 16):
                    sl = pl.ds(16 * q, 16)
                    atom_v[r, sl] = (pk[q] * hrows_v[buf, r, sl]
                                     * trows_v[buf, r, sl])
                return carry

            lax.fori_loop(0, CHUNK, rbody, 0, unroll=4)
            pltpu.sync_copy(atom_v,
                            out_hbm.at[pl.ds(base + c * CHUNK, CHUNK)])

    return k(table, pred2, heads2d, tails2d)


def _tc_transpose(tblT):
    BLKC = 16384
    n_blk = (N_ROWS + BLKC - 1) // BLKC

    def body(a_ref, o_ref):
        o_ref[...] = a_ref[...].T

    return pl.pallas_call(
        body,
        grid=(n_blk,),
        in_specs=[pl.BlockSpec((D, BLKC), lambda i: (0, i))],
        out_specs=pl.BlockSpec((BLKC, D), lambda i: (i, 0)),
        out_shape=jax.ShapeDtypeStruct((N_ROWS, D), jnp.float32),
    )(tblT)


def _tc_matmul(atom, W, b2):
    BLK = 4096
    n_blk = T // BLK

    def body(a_ref, w_ref, b_ref, o_ref):
        o_ref[...] = jnp.dot(a_ref[...], w_ref[...],
                             preferred_element_type=jnp.float32) + b_ref[...]

    return pl.pallas_call(
        body,
        grid=(n_blk,),
        in_specs=[
            pl.BlockSpec((BLK, D), lambda i: (i, 0)),
            pl.BlockSpec((D, D), lambda i: (0, 0)),
            pl.BlockSpec((1, D), lambda i: (0, 0)),
        ],
        out_specs=pl.BlockSpec((BLK, D), lambda i: (i, 0)),
        out_shape=jax.ShapeDtypeStruct((T, D), jnp.float32),
    )(atom, W, b2)


def kernel(constant_emb, predicate_emb, W, b, indices_p0, indices_p1):
    heads = jnp.concatenate([indices_p0[:, 0], indices_p1[:, 0]], axis=0)
    tails = jnp.concatenate([indices_p0[:, 1], indices_p1[:, 1]], axis=0)
    heads2d = heads.astype(jnp.int32).reshape(T // 128, 128)
    tails2d = tails.astype(jnp.int32).reshape(T // 128, 128)
    pred2 = predicate_emb[:2]
    table_rm = _tc_transpose(constant_emb.T)
    atom = _sc_atom(table_rm, pred2, heads2d, tails2d)
    return _tc_matmul(atom, W, b.reshape(1, D))


# speedup vs baseline: 1.1408x; 1.1408x over previous
import functools

import jax
import jax.numpy as jnp
from jax import lax
from jax.experimental import pallas as pl
from jax.experimental.pallas import tpu as pltpu
from jax.experimental.pallas import tpu_sc as plsc

D = 64
PW = 128
N_TRIP = 16384
T = 2 * N_TRIP

NC, NS = 2, 16
NW = NC * NS
TRIP_PER_W = T // NW
CHUNK = 128
N_CHUNK = TRIP_PER_W // CHUNK
STREAMS = CHUNK // 128
NBUF = 2


def _sc_atom(table, pred2, heads2d, tails2d):
    mesh = plsc.VectorSubcoreMesh(core_axis_name="c", subcore_axis_name="s")
    idx_rows = TRIP_PER_W // 128

    @functools.partial(
        pl.kernel,
        out_type=jax.ShapeDtypeStruct((T, D), jnp.float32),
        mesh=mesh,
        scratch_types=[
            pltpu.VMEM((idx_rows, 128), jnp.int32),
            pltpu.VMEM((idx_rows, 128), jnp.int32),
            pltpu.VMEM((D,), jnp.float32),
            pltpu.VMEM((NBUF, CHUNK, PW), jnp.float32),
            pltpu.VMEM((NBUF, CHUNK, PW), jnp.float32),
            pltpu.VMEM((CHUNK, D), jnp.float32),
            pltpu.SemaphoreType.DMA,
            pltpu.SemaphoreType.DMA,
        ],
        compiler_params=pltpu.CompilerParams(use_tc_tiling_on_sc=False),
    )
    def k(table_hbm, pred_hbm, heads_hbm, tails_hbm, out_hbm,
          hidx_v, tidx_v, pred_v, hrows_v, trows_v, atom_v, hsem, tsem):
        wid = lax.axis_index("s") * NC + lax.axis_index("c")
        base = wid * TRIP_PER_W
        pltpu.sync_copy(heads_hbm.at[pl.ds(wid * idx_rows, idx_rows)], hidx_v)
        pltpu.sync_copy(tails_hbm.at[pl.ds(wid * idx_rows, idx_rows)], tidx_v)
        pltpu.sync_copy(pred_hbm.at[wid // NS], pred_v)
        pk = [pred_v[pl.ds(16 * q, 16)] for q in range(D // 16)]

        def fire(c, buf):
            for j in range(STREAMS):
                pltpu.async_copy(
                    table_hbm.at[hidx_v.at[c * STREAMS + j]],
                    hrows_v.at[buf].at[pl.ds(j * 128, 128)], hsem)
                pltpu.async_copy(
                    table_hbm.at[tidx_v.at[c * STREAMS + j]],
                    trows_v.at[buf].at[pl.ds(j * 128, 128)], tsem)

        def drain(c, buf):
            for j in range(STREAMS):
                pltpu.make_async_copy(
                    table_hbm.at[hidx_v.at[c * STREAMS + j]],
                    hrows_v.at[buf].at[pl.ds(j * 128, 128)], hsem).wait()
                pltpu.make_async_copy(
                    table_hbm.at[tidx_v.at[c * STREAMS + j]],
                    trows_v.at[buf].at[pl.ds(j * 128, 128)], tsem).wait()

        fire(0, 0)
        for c in range(N_CHUNK):
            buf = c % NBUF
            drain(c, buf)
            if c + 1 < N_CHUNK:
                fire(c + 1, (c + 1) % NBUF)

            def rbody(r, carry):
                for q in range(D // 16):
                    sl = pl.ds(16 * q, 16)
                    atom_v[r, sl] = (pk[q] * hrows_v[buf, r, sl]
                                     * trows_v[buf, r, sl])
                return carry

            lax.fori_loop(0, CHUNK, rbody, 0, unroll=4)
            pltpu.sync_copy(atom_v,
                            out_hbm.at[pl.ds(base + c * CHUNK, CHUNK)])

    return k(table, pred2, heads2d, tails2d)


def _tc_matmul(atom, W, b2):
    BLK = 4096
    n_blk = T // BLK

    def body(a_ref, w_ref, b_ref, o_ref):
        o_ref[...] = jnp.dot(a_ref[...], w_ref[...],
                             preferred_element_type=jnp.float32) + b_ref[...]

    return pl.pallas_call(
        body,
        grid=(n_blk,),
        in_specs=[
            pl.BlockSpec((BLK, D), lambda i: (i, 0)),
            pl.BlockSpec((D, D), lambda i: (0, 0)),
            pl.BlockSpec((1, D), lambda i: (0, 0)),
        ],
        out_specs=pl.BlockSpec((BLK, D), lambda i: (i, 0)),
        out_shape=jax.ShapeDtypeStruct((T, D), jnp.float32),
    )(atom, W, b2)


def kernel(constant_emb, predicate_emb, W, b, indices_p0, indices_p1):
    heads = jnp.concatenate([indices_p0[:, 0], indices_p1[:, 0]], axis=0)
    tails = jnp.concatenate([indices_p0[:, 1], indices_p1[:, 1]], axis=0)
    heads2d = heads.astype(jnp.int32).reshape(T // 128, 128)
    tails2d = tails.astype(jnp.int32).reshape(T // 128, 128)
    pred2 = predicate_emb[:2]
    tbl_pad = jnp.pad(constant_emb, ((0, 0), (0, PW - D)))
    atom = _sc_atom(tbl_pad, pred2, heads2d, tails2d)
    return _tc_matmul(atom, W, b.reshape(1, D))


# speedup vs baseline: 1.1650x; 1.0212x over previous
import functools

import jax
import jax.numpy as jnp
from jax import lax
from jax.experimental import pallas as pl
from jax.experimental.pallas import tpu as pltpu
from jax.experimental.pallas import tpu_sc as plsc

D = 64
PW = 128
N_TRIP = 16384
T = 2 * N_TRIP

NC, NS = 2, 16
NW = NC * NS
TRIP_PER_W = T // NW
CHUNK = 128
N_CHUNK = TRIP_PER_W // CHUNK
STREAMS = CHUNK // 128
NBUF = 2


def _sc_atom(table, pred2, heads2d, tails2d):
    mesh = plsc.VectorSubcoreMesh(core_axis_name="c", subcore_axis_name="s")
    idx_rows = TRIP_PER_W // 128

    @functools.partial(
        pl.kernel,
        out_type=jax.ShapeDtypeStruct((T, D), jnp.float32),
        mesh=mesh,
        scratch_types=[
            pltpu.VMEM((idx_rows, 128), jnp.int32),
            pltpu.VMEM((idx_rows, 128), jnp.int32),
            pltpu.VMEM((D,), jnp.float32),
            pltpu.VMEM((NBUF, CHUNK, PW), jnp.float32),
            pltpu.VMEM((NBUF, CHUNK, PW), jnp.float32),
            pltpu.VMEM((CHUNK, D), jnp.float32),
            pltpu.SemaphoreType.DMA,
            pltpu.SemaphoreType.DMA,
        ],
        compiler_params=pltpu.CompilerParams(use_tc_tiling_on_sc=False),
    )
    def k(table_hbm, pred_hbm, heads_hbm, tails_hbm, out_hbm,
          hidx_v, tidx_v, pred_v, hrows_v, trows_v, atom_v, hsem, tsem):
        wid = lax.axis_index("s") * NC + lax.axis_index("c")
        base = wid * TRIP_PER_W
        pltpu.sync_copy(heads_hbm.at[pl.ds(wid * idx_rows, idx_rows)], hidx_v)
        pltpu.sync_copy(tails_hbm.at[pl.ds(wid * idx_rows, idx_rows)], tidx_v)
        pltpu.sync_copy(pred_hbm.at[wid // NS], pred_v)
        pk = [pred_v[pl.ds(16 * q, 16)] for q in range(D // 16)]

        def fire(c, buf):
            for j in range(STREAMS):
                pltpu.async_copy(
                    table_hbm.at[hidx_v.at[c * STREAMS + j]],
                    hrows_v.at[buf].at[pl.ds(j * 128, 128)], hsem)
                pltpu.async_copy(
                    table_hbm.at[tidx_v.at[c * STREAMS + j]],
                    trows_v.at[buf].at[pl.ds(j * 128, 128)], tsem)

        def drain(c, buf):
            for j in range(STREAMS):
                pltpu.make_async_copy(
                    table_hbm.at[hidx_v.at[c * STREAMS + j]],
                    hrows_v.at[buf].at[pl.ds(j * 128, 128)], hsem).wait()
                pltpu.make_async_copy(
                    table_hbm.at[tidx_v.at[c * STREAMS + j]],
                    trows_v.at[buf].at[pl.ds(j * 128, 128)], tsem).wait()

        fire(0, 0)
        for c in range(N_CHUNK):
            buf = c % NBUF
            drain(c, buf)
            if c + 1 < N_CHUNK:
                fire(c + 1, (c + 1) % NBUF)

            def rbody(r, carry):
                for q in range(D // 16):
                    sl = pl.ds(16 * q, 16)
                    atom_v[r, sl] = (pk[q] * hrows_v[buf, r, sl]
                                     * trows_v[buf, r, sl])
                return carry

            lax.fori_loop(0, CHUNK, rbody, 0, unroll=4)
            pltpu.sync_copy(atom_v,
                            out_hbm.at[pl.ds(base + c * CHUNK, CHUNK)])

    return k(table, pred2, heads2d, tails2d)


def _tc_matmul(atom, W, b2):
    BLK = 4096
    n_blk = T // BLK

    def body(a_ref, w_ref, b_ref, o_ref):
        o_ref[...] = lax.dot_general(
            w_ref[...], a_ref[...], (((0,), (1,)), ((), ())),
            preferred_element_type=jnp.float32) + b_ref[...]

    return pl.pallas_call(
        body,
        grid=(n_blk,),
        in_specs=[
            pl.BlockSpec((BLK, D), lambda i: (i, 0)),
            pl.BlockSpec((D, D), lambda i: (0, 0)),
            pl.BlockSpec((D, 1), lambda i: (0, 0)),
        ],
        out_specs=pl.BlockSpec((D, BLK), lambda i: (0, i)),
        out_shape=jax.ShapeDtypeStruct((D, T), jnp.float32),
    )(atom, W, b2)


def kernel(constant_emb, predicate_emb, W, b, indices_p0, indices_p1):
    heads = jnp.concatenate([indices_p0[:, 0], indices_p1[:, 0]], axis=0)
    tails = jnp.concatenate([indices_p0[:, 1], indices_p1[:, 1]], axis=0)
    heads2d = heads.astype(jnp.int32).reshape(T // 128, 128)
    tails2d = tails.astype(jnp.int32).reshape(T // 128, 128)
    pred2 = predicate_emb[:2]
    tbl_pad = jnp.pad(constant_emb, ((0, 0), (0, PW - D)))
    atom = _sc_atom(tbl_pad, pred2, heads2d, tails2d)
    return _tc_matmul(atom, W, b.reshape(D, 1)).T
